# Initial kernel scaffold; baseline (speedup 1.0000x reference)
import jax
import jax.numpy as jnp
from jax.experimental import pallas as pl


def kernel(x, edge_index, mask_weights):
    raise NotImplementedError("write your pallas kernel here")


# speedup vs baseline: 1.7074x; 1.7074x over previous
import functools

import jax
import jax.numpy as jnp
from jax import lax
from jax.experimental import pallas as pl
from jax.experimental.pallas import tpu as pltpu
from jax.experimental.pallas import tpu_sc as plsc

_N_NODES = 100000
_EMB = 128
_N_EDGES = 1600000
_NPAD = 100352
_NC, _NS = 2, 16
_NW = _NC * _NS
_EROWS = 12544
_ROWS_PER_TILE = _EROWS // _NW
_K = 8
_CHUNKS = _ROWS_PER_TILE // _K
_SLICE = _NPAD // _NS


def _sc_presence_flags(src2d, dst2d):
    mesh = plsc.VectorSubcoreMesh(core_axis_name="c", subcore_axis_name="s")

    @functools.partial(
        pl.kernel,
        out_type=jax.ShapeDtypeStruct((_NC, 2, _NPAD), jnp.int32),
        mesh=mesh,
        scratch_types=[
            pltpu.VMEM_SHARED((_NPAD,), jnp.int32),
            pltpu.VMEM_SHARED((_NPAD,), jnp.int32),
            pltpu.VMEM((_K, 128), jnp.int32),
            pltpu.VMEM((_K, 128), jnp.int32),
            pltpu.VMEM((128,), jnp.int32),
            pltpu.VMEM((_SLICE,), jnp.int32),
            pltpu.SemaphoreType.DMA,
            pltpu.SemaphoreType.DMA,
        ],
    )
    def k(src_hbm, dst_hbm, out_hbm, out_fl, in_fl, idx_s, idx_d, ones_v,
          zeros_v, sem_st, sem_sc):
        cid = lax.axis_index("c")
        sid = lax.axis_index("s")
        wid = sid * _NC + cid

        def fill_ones(i, _):
            ones_v[pl.ds(i * 16, 16)] = jnp.ones((16,), jnp.int32)
            return 0

        lax.fori_loop(0, 128 // 16, fill_ones, 0)

        def fill_zeros(i, _):
            zeros_v[pl.ds(i * 16, 16)] = jnp.zeros((16,), jnp.int32)
            return 0

        lax.fori_loop(0, _SLICE // 16, fill_zeros, 0)

        pltpu.sync_copy(zeros_v, out_fl.at[pl.ds(sid * _SLICE, _SLICE)])
        pltpu.sync_copy(zeros_v, in_fl.at[pl.ds(sid * _SLICE, _SLICE)])
        plsc.subcore_barrier()

        row0 = wid * _ROWS_PER_TILE

        def chunk(ci, _):
            base = row0 + ci * _K
            c1 = pltpu.async_copy(src_hbm.at[pl.ds(base, _K)], idx_s, sem_st)
            c2 = pltpu.async_copy(dst_hbm.at[pl.ds(base, _K)], idx_d, sem_st)
            c1.wait()
            c2.wait()
            cps = []
            for j in range(_K):
                cps.append(pltpu.async_copy(ones_v, out_fl.at[idx_s.at[j]], sem_sc))
                cps.append(pltpu.async_copy(ones_v, in_fl.at[idx_d.at[j]], sem_sc))
            for cp in cps:
                cp.wait()
            return 0

        lax.fori_loop(0, _CHUNKS, chunk, 0)
        plsc.subcore_barrier()

        sl = pl.ds(sid * _SLICE, _SLICE)
        pltpu.sync_copy(out_fl.at[sl], out_hbm.at[cid, 0, sl])
        pltpu.sync_copy(in_fl.at[sl], out_hbm.at[cid, 1, sl])

    return k(src2d, dst2d)


_TC_B = 5000


def _tc_apply(flags_t, w, x2d):
    def body(f_ref, w_ref, x_ref, o_ref):
        f = f_ref[...]
        outp = (f[:, 0:1] + f[:, 1:2]) > 0
        inp = (f[:, 2:3] + f[:, 3:4]) > 0
        cat = 2 * inp.astype(jnp.int32) + 1 - outp.astype(jnp.int32)
        wv = w_ref[...]
        m = jnp.where(
            cat == 0, wv[0:1, :],
            jnp.where(cat == 1, wv[1:2, :],
                      jnp.where(cat == 2, wv[2:3, :], wv[3:4, :])))
        o_ref[...] = x_ref[...] * m

    return pl.pallas_call(
        body,
        grid=(_N_NODES // _TC_B,),
        in_specs=[
            pl.BlockSpec((_TC_B, 4), lambda i: (i, 0)),
            pl.BlockSpec((4, _EMB), lambda i: (0, 0)),
            pl.BlockSpec((_TC_B, _EMB), lambda i: (i, 0)),
        ],
        out_specs=pl.BlockSpec((_TC_B, _EMB), lambda i: (i, 0)),
        out_shape=jax.ShapeDtypeStruct((_N_NODES, _EMB), jnp.float32),
    )(flags_t, w, x2d)


def kernel(x, edge_index, mask_weights):
    src = edge_index[0].astype(jnp.int32)
    dst = edge_index[1].astype(jnp.int32)
    pad = jnp.full((_EROWS * 128 - _N_EDGES,), _N_NODES, jnp.int32)
    src2d = jnp.concatenate([src, pad]).reshape(_EROWS, 128)
    dst2d = jnp.concatenate([dst, pad]).reshape(_EROWS, 128)

    flags = _sc_presence_flags(src2d, dst2d)
    flags_t = flags.transpose(1, 0, 2).reshape(4, _NPAD)[:, :_N_NODES].T

    out = _tc_apply(flags_t, mask_weights, x[0])
    return out.reshape(1, _N_NODES, _EMB)


# speedup vs baseline: 2.3218x; 1.3599x over previous
import functools

import jax
import jax.numpy as jnp
from jax import lax
from jax.experimental import pallas as pl
from jax.experimental.pallas import tpu as pltpu
from jax.experimental.pallas import tpu_sc as plsc

_N_NODES = 100000
_EMB = 128
_N_EDGES = 1600000
_NPAD = 100352
_NC, _NS = 2, 16
_NW = _NC * _NS
_EROWS = 12500
_MAIN_ROWS = 12496
_ROWS_PER_TILE = 392
_K = 8
_CHUNKS = 49
_SLICE = _NPAD // _NS


def _sc_presence_flags(edge3d, edge_tail):
    mesh = plsc.VectorSubcoreMesh(core_axis_name="c", subcore_axis_name="s")

    @functools.partial(
        pl.kernel,
        out_type=jax.ShapeDtypeStruct((_NC, 2, _NPAD), jnp.int32),
        mesh=mesh,
        scratch_types=[
            pltpu.VMEM_SHARED((_NPAD,), jnp.int32),
            pltpu.VMEM_SHARED((_NPAD,), jnp.int32),
            pltpu.VMEM((_K, 128), jnp.int32),
            pltpu.VMEM((_K, 128), jnp.int32),
            pltpu.VMEM((128,), jnp.int32),
            pltpu.VMEM((_SLICE,), jnp.int32),
            pltpu.SemaphoreType.DMA,
            pltpu.SemaphoreType.DMA,
        ],
    )
    def k(edge_hbm, tail_hbm, out_hbm, out_fl, in_fl, idx_s, idx_d, ones_v,
          zeros_v, sem_st, sem_sc):
        cid = lax.axis_index("c")
        sid = lax.axis_index("s")
        wid = sid * _NC + cid

        def fill_ones(i, _):
            ones_v[pl.ds(i * 16, 16)] = jnp.ones((16,), jnp.int32)
            return 0

        lax.fori_loop(0, 128 // 16, fill_ones, 0)

        def fill_zeros(i, _):
            zeros_v[pl.ds(i * 16, 16)] = jnp.zeros((16,), jnp.int32)
            return 0

        lax.fori_loop(0, _SLICE // 16, fill_zeros, 0)

        pltpu.sync_copy(zeros_v, out_fl.at[pl.ds(sid * _SLICE, _SLICE)])
        pltpu.sync_copy(zeros_v, in_fl.at[pl.ds(sid * _SLICE, _SLICE)])
        plsc.subcore_barrier()

        row0 = jnp.minimum(wid * _ROWS_PER_TILE, _MAIN_ROWS - _ROWS_PER_TILE)

        def chunk(ci, _):
            base = row0 + jnp.minimum(ci * _K, _ROWS_PER_TILE - _K)
            c1 = pltpu.async_copy(edge_hbm.at[0, pl.ds(base, _K)], idx_s, sem_st)
            c2 = pltpu.async_copy(edge_hbm.at[1, pl.ds(base, _K)], idx_d, sem_st)
            c1.wait()
            c2.wait()
            cps = []
            for j in range(_K):
                cps.append(pltpu.async_copy(ones_v, out_fl.at[idx_s.at[j]], sem_sc))
                cps.append(pltpu.async_copy(ones_v, in_fl.at[idx_d.at[j]], sem_sc))
            for cp in cps:
                cp.wait()
            return 0

        lax.fori_loop(0, _CHUNKS, chunk, 0)

        @pl.when(wid == 0)
        def _tail():
            c1 = pltpu.async_copy(tail_hbm.at[0], idx_s.at[pl.ds(0, 4)], sem_st)
            c2 = pltpu.async_copy(tail_hbm.at[1], idx_d.at[pl.ds(0, 4)], sem_st)
            c1.wait()
            c2.wait()
            cps = []
            for j in range(4):
                cps.append(pltpu.async_copy(ones_v, out_fl.at[idx_s.at[j]], sem_sc))
                cps.append(pltpu.async_copy(ones_v, in_fl.at[idx_d.at[j]], sem_sc))
            for cp in cps:
                cp.wait()

        plsc.subcore_barrier()

        sl = pl.ds(sid * _SLICE, _SLICE)
        pltpu.sync_copy(out_fl.at[sl], out_hbm.at[cid, 0, sl])
        pltpu.sync_copy(in_fl.at[sl], out_hbm.at[cid, 1, sl])

    return k(edge3d, edge_tail)


_TC_B = 5000


def _tc_apply(flags_t, w, x2d):
    def body(f_ref, w_ref, x_ref, o_ref):
        f = f_ref[...].astype(jnp.int32)
        has_out = (f[:, 0:1] + f[:, 1:2]) > 0
        has_in = (f[:, 2:3] + f[:, 3:4]) > 0
        wv = w_ref[...]
        m = jnp.where(
            has_in,
            jnp.where(has_out, wv[2:3, :], wv[3:4, :]),
            jnp.where(has_out, wv[0:1, :], wv[1:2, :]))
        o_ref[...] = x_ref[...] * m

    return pl.pallas_call(
        body,
        grid=(_N_NODES // _TC_B,),
        in_specs=[
            pl.BlockSpec((_TC_B, 4), lambda i: (i, 0)),
            pl.BlockSpec((4, _EMB), lambda i: (0, 0)),
            pl.BlockSpec((_TC_B, _EMB), lambda i: (i, 0)),
        ],
        out_specs=pl.BlockSpec((_TC_B, _EMB), lambda i: (i, 0)),
        out_shape=jax.ShapeDtypeStruct((_N_NODES, _EMB), jnp.float32),
    )(flags_t, w, x2d)


def kernel(x, edge_index, mask_weights):
    e32 = edge_index.astype(jnp.int32)
    edge3d = e32.reshape(2, _EROWS, 128)
    edge_tail = e32[:, _MAIN_ROWS * 128:].reshape(2, 4, 128)

    flags = _sc_presence_flags(edge3d, edge_tail)
    flags_t = (flags.astype(jnp.int8)
               .transpose(1, 0, 2).reshape(4, _NPAD)[:, :_N_NODES].T)

    out = _tc_apply(flags_t, mask_weights, x[0])
    return out.reshape(1, _N_NODES, _EMB)


# speedup vs baseline: 2.6030x; 1.1211x over previous
import functools

import jax
import jax.numpy as jnp
from jax import lax
from jax.experimental import pallas as pl
from jax.experimental.pallas import tpu as pltpu
from jax.experimental.pallas import tpu_sc as plsc

_N_NODES = 100000
_EMB = 128
_N_EDGES = 1600000
_NPAD = 100352
_NC, _NS = 2, 16
_NW = _NC * _NS
_EPT = _N_EDGES // _NW
_CHUNK = 4096
_CHUNKS = 13
_SLICE = _NPAD // _NS


def _sc_presence_flags(edge1d):
    mesh = plsc.VectorSubcoreMesh(core_axis_name="c", subcore_axis_name="s")

    @functools.partial(
        pl.kernel,
        out_type=jax.ShapeDtypeStruct((_NC, 2, _NPAD), jnp.int32),
        mesh=mesh,
        scratch_types=[
            pltpu.VMEM_SHARED((_NPAD,), jnp.int32),
            pltpu.VMEM_SHARED((_NPAD,), jnp.int32),
            pltpu.VMEM((_CHUNK,), jnp.int32),
            pltpu.VMEM((_CHUNK,), jnp.int32),
            pltpu.VMEM((_CHUNK,), jnp.int32),
            pltpu.VMEM((_SLICE,), jnp.int32),
            pltpu.SemaphoreType.DMA,
            pltpu.SemaphoreType.DMA,
        ],
    )
    def k(edge_hbm, out_hbm, out_fl, in_fl, idx_s, idx_d, ones_v,
          zeros_v, sem_st, sem_sc):
        cid = lax.axis_index("c")
        sid = lax.axis_index("s")
        wid = sid * _NC + cid

        def fill_ones(i, _):
            ones_v[pl.ds(i * 16, 16)] = jnp.ones((16,), jnp.int32)
            return 0

        lax.fori_loop(0, _CHUNK // 16, fill_ones, 0)

        def fill_zeros(i, _):
            zeros_v[pl.ds(i * 16, 16)] = jnp.zeros((16,), jnp.int32)
            return 0

        lax.fori_loop(0, _SLICE // 16, fill_zeros, 0)

        pltpu.sync_copy(zeros_v, out_fl.at[pl.ds(sid * _SLICE, _SLICE)])
        pltpu.sync_copy(zeros_v, in_fl.at[pl.ds(sid * _SLICE, _SLICE)])
        plsc.subcore_barrier()

        base0 = wid * _EPT

        def chunk(ci, _):
            base = base0 + jnp.minimum(ci * _CHUNK, _EPT - _CHUNK)
            c1 = pltpu.async_copy(edge_hbm.at[pl.ds(base, _CHUNK)], idx_s, sem_st)
            c2 = pltpu.async_copy(edge_hbm.at[pl.ds(_N_EDGES + base, _CHUNK)],
                                  idx_d, sem_st)
            c1.wait()
            c2.wait()
            s1 = pltpu.async_copy(ones_v, out_fl.at[idx_s], sem_sc)
            s2 = pltpu.async_copy(ones_v, in_fl.at[idx_d], sem_sc)
            s1.wait()
            s2.wait()
            return 0

        lax.fori_loop(0, _CHUNKS, chunk, 0)
        plsc.subcore_barrier()

        sl = pl.ds(sid * _SLICE, _SLICE)
        pltpu.sync_copy(out_fl.at[sl], out_hbm.at[cid, 0, sl])
        pltpu.sync_copy(in_fl.at[sl], out_hbm.at[cid, 1, sl])

    return k(edge1d)


_TC_B = 5000


def _tc_apply(flags_t, w, x2d):
    def body(f_ref, w_ref, x_ref, o_ref):
        f = f_ref[...].astype(jnp.int32)
        has_out = (f[:, 0:1] + f[:, 1:2]) > 0
        has_in = (f[:, 2:3] + f[:, 3:4]) > 0
        wv = w_ref[...]
        m = jnp.where(
            has_in,
            jnp.where(has_out, wv[2:3, :], wv[3:4, :]),
            jnp.where(has_out, wv[0:1, :], wv[1:2, :]))
        o_ref[...] = x_ref[...] * m

    return pl.pallas_call(
        body,
        grid=(_N_NODES // _TC_B,),
        in_specs=[
            pl.BlockSpec((_TC_B, 4), lambda i: (i, 0)),
            pl.BlockSpec((4, _EMB), lambda i: (0, 0)),
            pl.BlockSpec((_TC_B, _EMB), lambda i: (i, 0)),
        ],
        out_specs=pl.BlockSpec((_TC_B, _EMB), lambda i: (i, 0)),
        out_shape=jax.ShapeDtypeStruct((_N_NODES, _EMB), jnp.float32),
    )(flags_t, w, x2d)


def kernel(x, edge_index, mask_weights):
    edge1d = edge_index.astype(jnp.int32).reshape(2 * _N_EDGES)

    flags = _sc_presence_flags(edge1d)
    flags_t = (flags.astype(jnp.int8)
               .transpose(1, 0, 2).reshape(4, _NPAD)[:, :_N_NODES].T)

    out = _tc_apply(flags_t, mask_weights, x[0])
    return out.reshape(1, _N_NODES, _EMB)


# speedup vs baseline: 2.8963x; 1.1127x over previous
import functools

import jax
import jax.numpy as jnp
from jax import lax
from jax.experimental import pallas as pl
from jax.experimental.pallas import tpu as pltpu
from jax.experimental.pallas import tpu_sc as plsc

_N_NODES = 100000
_EMB = 128
_N_EDGES = 1600000
_NPAD = 100352
_NC, _NS = 2, 16
_NW = _NC * _NS
_EPT = _N_EDGES // _NW
_CHUNK = 3584
_CHUNKS = 14
_SLICE = _NPAD // _NS


def _sc_presence_flags(edge1d):
    mesh = plsc.VectorSubcoreMesh(core_axis_name="c", subcore_axis_name="s")

    @functools.partial(
        pl.kernel,
        out_type=jax.ShapeDtypeStruct((_NC, 2, _NPAD), jnp.int32),
        mesh=mesh,
        scratch_types=[
            pltpu.VMEM_SHARED((_NPAD,), jnp.int32),
            pltpu.VMEM_SHARED((_NPAD,), jnp.int32),
            pltpu.VMEM((_CHUNK,), jnp.int32),
            pltpu.VMEM((_CHUNK,), jnp.int32),
            pltpu.VMEM((_CHUNK,), jnp.int32),
            pltpu.VMEM((_CHUNK,), jnp.int32),
            pltpu.VMEM((_CHUNK,), jnp.int32),
            pltpu.VMEM((_SLICE,), jnp.int32),
            pltpu.SemaphoreType.DMA,
            pltpu.SemaphoreType.DMA,
        ],
    )
    def k(edge_hbm, out_hbm, out_fl, in_fl, idx_s0, idx_d0, idx_s1, idx_d1,
          ones_v, zeros_v, sem_st, sem_sc):
        cid = lax.axis_index("c")
        sid = lax.axis_index("s")
        wid = sid * _NC + cid

        def fill_ones(i, _):
            ones_v[pl.ds(i * 16, 16)] = jnp.ones((16,), jnp.int32)
            return 0

        lax.fori_loop(0, _CHUNK // 16, fill_ones, 0)

        def fill_zeros(i, _):
            zeros_v[pl.ds(i * 16, 16)] = jnp.zeros((16,), jnp.int32)
            return 0

        lax.fori_loop(0, _SLICE // 16, fill_zeros, 0)

        pltpu.sync_copy(zeros_v, out_fl.at[pl.ds(sid * _SLICE, _SLICE)])
        pltpu.sync_copy(zeros_v, in_fl.at[pl.ds(sid * _SLICE, _SLICE)])
        plsc.subcore_barrier()

        base0 = wid * _EPT

        def stage(ci, bs, bd):
            base = base0 + jnp.minimum(ci * _CHUNK, _EPT - _CHUNK)
            pltpu.async_copy(edge_hbm.at[pl.ds(base, _CHUNK)], bs, sem_st)
            pltpu.async_copy(edge_hbm.at[pl.ds(_N_EDGES + base, _CHUNK)], bd,
                             sem_st)

        def wait_stage(bs, bd):
            pltpu.make_async_copy(edge_hbm.at[pl.ds(0, _CHUNK)], bs, sem_st).wait()
            pltpu.make_async_copy(edge_hbm.at[pl.ds(0, _CHUNK)], bd, sem_st).wait()

        stage(0, idx_s0, idx_d0)

        def it_body(it, _):
            bufs = ((idx_s0, idx_d0, idx_s1, idx_d1),
                    (idx_s1, idx_d1, idx_s0, idx_d0))
            for b in range(2):
                bs, bd, ns, nd = bufs[b]
                ci = 2 * it + b
                wait_stage(bs, bd)
                stage(ci + 1, ns, nd)
                s1 = pltpu.async_copy(ones_v, out_fl.at[bs], sem_sc)
                s2 = pltpu.async_copy(ones_v, in_fl.at[bd], sem_sc)
                s1.wait()
                s2.wait()
            return 0

        lax.fori_loop(0, _CHUNKS // 2, it_body, 0)
        wait_stage(idx_s0, idx_d0)
        plsc.subcore_barrier()

        sl = pl.ds(sid * _SLICE, _SLICE)
        pltpu.sync_copy(out_fl.at[sl], out_hbm.at[cid, 0, sl])
        pltpu.sync_copy(in_fl.at[sl], out_hbm.at[cid, 1, sl])

    return k(edge1d)


_TC_B = 10000


def _tc_apply(flags_t, w, x2d):
    def body(f_ref, w_ref, x_ref, o_ref):
        f = f_ref[...].astype(jnp.int32)
        has_out = (f[:, 0:1] + f[:, 1:2]) > 0
        has_in = (f[:, 2:3] + f[:, 3:4]) > 0
        wv = w_ref[...]
        m = jnp.where(
            has_in,
            jnp.where(has_out, wv[2:3, :], wv[3:4, :]),
            jnp.where(has_out, wv[0:1, :], wv[1:2, :]))
        o_ref[...] = x_ref[...] * m

    return pl.pallas_call(
        body,
        grid=(_N_NODES // _TC_B,),
        in_specs=[
            pl.BlockSpec((_TC_B, 4), lambda i: (i, 0)),
            pl.BlockSpec((4, _EMB), lambda i: (0, 0)),
            pl.BlockSpec((_TC_B, _EMB), lambda i: (i, 0)),
        ],
        out_specs=pl.BlockSpec((_TC_B, _EMB), lambda i: (i, 0)),
        out_shape=jax.ShapeDtypeStruct((_N_NODES, _EMB), jnp.float32),
    )(flags_t, w, x2d)


def kernel(x, edge_index, mask_weights):
    edge1d = edge_index.astype(jnp.int32).reshape(2 * _N_EDGES)

    flags = _sc_presence_flags(edge1d)
    flags_t = (flags.astype(jnp.int8)
               .transpose(1, 0, 2).reshape(4, _NPAD)[:, :_N_NODES].T)

    out = _tc_apply(flags_t, mask_weights, x[0])
    return out.reshape(1, _N_NODES, _EMB)
